# baseline (speedup 1.0000x reference)
import functools

import jax
import jax.numpy as jnp
from jax import lax
from jax.experimental import pallas as pl
from jax.experimental.pallas import tpu as pltpu
from jax.experimental.pallas import tpu_sc as plsc

DIM_ = 1024
HID_ = 2048
E_ = 8
R_ = 8
N_ = 2048
A_ = 2 * N_
TM_ = 512
SPAD_ = A_ + E_ * TM_
NT_ = SPAD_ // TM_
NC_ = 2
NS_ = 16
NW_ = NC_ * NS_

_CHUNK = 512


def _route_body(x_ref, wr_ref, br_ref, pos_ref, te_ref, w_ref, x16_ref):
    x = x_ref[...]
    x16_ref[...] = x.astype(jnp.bfloat16)
    wr = wr_ref[...]
    logits = lax.dot_general(wr, x, (((1,), (1,)), ((), ())),
                             preferred_element_type=jnp.float32)
    logits = logits + br_ref[...].reshape(E_, 1)
    iota_e = lax.broadcasted_iota(jnp.int32, (E_, N_), 0)
    neg_inf = jnp.float32(-jnp.inf)
    m1 = jnp.max(logits, axis=0, keepdims=True)
    i1 = jnp.min(jnp.where(logits == m1, iota_e, E_), axis=0, keepdims=True)
    masked = jnp.where(iota_e == i1, neg_inf, logits)
    m2 = jnp.max(masked, axis=0, keepdims=True)
    i2 = jnp.min(jnp.where(masked == m2, iota_e, E_), axis=0, keepdims=True)
    t = jnp.exp(m2 - m1)
    w1 = 1.0 / (1.0 + t)
    w2 = 1.0 - w1
    wcat = jnp.concatenate([w1, w2], axis=1)
    w_ref[...] = lax.dot_general(wcat, jnp.ones((1, 16), jnp.float32),
                                 (((0,), (0,)), ((), ())),
                                 preferred_element_type=jnp.float32)

    ef = jnp.concatenate([i1, i2], axis=1)
    hot = (lax.broadcasted_iota(jnp.int32, (E_, A_), 0) == ef
           ).astype(jnp.float32)
    tri = (lax.broadcasted_iota(jnp.int32, (_CHUNK, _CHUNK), 0)
           <= lax.broadcasted_iota(jnp.int32, (_CHUNK, _CHUNK), 1)
           ).astype(jnp.float32)
    carry = jnp.zeros((E_, 1), jnp.float32)
    chunks = []
    for c in range(A_ // _CHUNK):
        hc = lax.slice(hot, (0, c * _CHUNK), (E_, (c + 1) * _CHUNK))
        cc = lax.dot_general(hc, tri, (((1,), (0,)), ((), ())),
                             preferred_element_type=jnp.float32) + carry
        carry = lax.slice(cc, (0, _CHUNK - 1), (E_, _CHUNK))
        chunks.append(cc)
    csum = jnp.concatenate(chunks, axis=1)
    counts = carry
    tilecnt = jnp.floor((counts + (TM_ - 1)) * (1.0 / TM_))
    mstrict = (lax.broadcasted_iota(jnp.int32, (E_, E_), 1)
               < lax.broadcasted_iota(jnp.int32, (E_, E_), 0)
               ).astype(jnp.float32)
    tstart = lax.dot_general(mstrict, tilecnt, (((1,), (0,)), ((), ())),
                             preferred_element_type=jnp.float32)
    rank = jnp.sum(hot * csum, axis=0, keepdims=True) - 1.0
    base = jnp.sum(hot * (tstart * TM_), axis=0, keepdims=True)
    pos_ref[...] = (base + rank).astype(jnp.int32)

    tend = tstart + tilecnt
    iota_m = lax.broadcasted_iota(jnp.int32, (E_, 128), 1).astype(jnp.float32)
    te = jnp.sum((iota_m >= tend).astype(jnp.float32), axis=0,
                 keepdims=True)
    n_used = jnp.sum(tilecnt)
    iota_1m = lax.broadcasted_iota(jnp.int32, (1, 128), 1).astype(jnp.float32)
    fi = jnp.minimum(iota_1m, n_used - 1.0)
    valid = (iota_1m < n_used).astype(jnp.float32)
    te_ref[...] = jnp.concatenate(
        [jnp.minimum(te, E_ - 1), fi, valid], axis=0).astype(jnp.int32)


def _route(x, Wr, br):
    return pl.pallas_call(
        _route_body,
        out_shape=(
            jax.ShapeDtypeStruct((1, A_), jnp.int32),
            jax.ShapeDtypeStruct((3, 128), jnp.int32),
            jax.ShapeDtypeStruct((A_, 16), jnp.float32),
            jax.ShapeDtypeStruct((N_, DIM_), jnp.bfloat16),
        ),
    )(x, Wr, br.reshape(1, E_))


def _dispatch_body(pos_hbm, x_hbm, xs_hbm, idx_v, rows_v, sem):
    c = lax.axis_index("c")
    s = lax.axis_index("s")
    wid = s * NC_ + c
    per_w = A_ // NW_
    sub = 32
    for k in range(per_w // sub):
        base = wid * per_w + k * sub
        tok = jnp.where(base >= N_, base - N_, base)
        pltpu.sync_copy(pos_hbm.at[pl.ds(base, sub)], idx_v)
        pltpu.sync_copy(x_hbm.at[pl.ds(tok, sub)], rows_v)
        pltpu.async_copy(rows_v, xs_hbm.at[idx_v], sem).wait()


def _dispatch(pos, x):
    mesh = plsc.VectorSubcoreMesh(core_axis_name="c", subcore_axis_name="s",
                                  num_cores=NC_, num_subcores=NS_)
    return pl.kernel(
        _dispatch_body,
        out_type=jax.ShapeDtypeStruct((SPAD_, DIM_ // 2), jnp.int32),
        mesh=mesh,
        scratch_types=[
            pltpu.VMEM((32,), jnp.int32),
            pltpu.VMEM((32, DIM_ // 2), jnp.int32),
            pltpu.SemaphoreType.DMA,
        ],
    )(pos, x)


def _ffn_body(te_ref, xs_ref, w1_ref, b1_ref, a1_ref, bb1_ref,
              w2_ref, b2_ref, a2_ref, bb2_ref, ys_ref,
              w1b_ref, w2b_ref, prev_ref):
    i = pl.program_id(0)
    e = te_ref[0, i]
    valid = te_ref[2, i] == 1

    nn = (((1,), (0,)), ((), ()))
    nt = (((1,), (1,)), ((), ()))

    @pl.when(valid & ((i == 0) | (e != prev_ref[0])))
    def _fold_weights():
        w1b_ref[...] = (w1_ref[0] + lax.dot_general(
            bb1_ref[0], a1_ref[0], nn, preferred_element_type=jnp.float32)
        ).astype(jnp.bfloat16)
        w2b_ref[...] = (w2_ref[0] + lax.dot_general(
            bb2_ref[0], a2_ref[0], nn, preferred_element_type=jnp.float32)
        ).astype(jnp.bfloat16)
        prev_ref[0] = e

    @pl.when(valid)
    def _compute():
        xb = xs_ref[...]
        u = lax.dot_general(xb, w1b_ref[...],
                            nt, preferred_element_type=jnp.float32)
        u = u + b1_ref[0]
        h = u * (0.5 + 0.5 * lax.erf(u * jnp.float32(0.7071067811865476)))
        o = lax.dot_general(h.astype(jnp.bfloat16), w2b_ref[...],
                            nt, preferred_element_type=jnp.float32)
        ys_ref[...] = o + b2_ref[0]


def _ffn(te, xs, W1, b1, A1, B1, W2, b2, A2, B2):
    def em(i, te_ref):
        return (te_ref[0, i], 0, 0)

    grid_spec = pltpu.PrefetchScalarGridSpec(
        num_scalar_prefetch=1,
        grid=(NT_,),
        in_specs=[
            pl.BlockSpec((TM_, DIM_), lambda i, te_ref: (te_ref[1, i], 0)),
            pl.BlockSpec((1, HID_, DIM_), em),
            pl.BlockSpec((1, 1, HID_), em),
            pl.BlockSpec((1, R_, DIM_), em),
            pl.BlockSpec((1, HID_, R_), em),
            pl.BlockSpec((1, DIM_, HID_), em),
            pl.BlockSpec((1, 1, DIM_), em),
            pl.BlockSpec((1, R_, HID_), em),
            pl.BlockSpec((1, DIM_, R_), em),
        ],
        out_specs=pl.BlockSpec((TM_, DIM_), lambda i, te_ref: (i, 0)),
        scratch_shapes=[
            pltpu.VMEM((HID_, DIM_), jnp.bfloat16),
            pltpu.VMEM((DIM_, HID_), jnp.bfloat16),
            pltpu.SMEM((1,), jnp.int32),
        ],
    )
    return pl.pallas_call(
        _ffn_body,
        grid_spec=grid_spec,
        out_shape=jax.ShapeDtypeStruct((SPAD_, DIM_), jnp.float32),
    )(te, xs, W1, b1.reshape(E_, 1, HID_), A1, B1, W2,
      b2.reshape(E_, 1, DIM_), A2, B2)


def _combine_body(pos_hbm, w_hbm, ys_hbm, y_hbm,
                  idx0_v, idx1_v, w0_v, w1_v, r0_v, r1_v, sem0, sem1):
    c = lax.axis_index("c")
    s = lax.axis_index("s")
    wid = s * NC_ + c
    per_w = N_ // NW_
    sub = 32
    for k in range(per_w // sub):
        tb = wid * per_w + k * sub
        pltpu.sync_copy(pos_hbm.at[pl.ds(tb, sub)], idx0_v)
        pltpu.sync_copy(pos_hbm.at[pl.ds(N_ + tb, sub)], idx1_v)
        pltpu.sync_copy(w_hbm.at[pl.ds(tb, sub)], w0_v)
        pltpu.sync_copy(w_hbm.at[pl.ds(N_ + tb, sub)], w1_v)
        cp0 = pltpu.async_copy(ys_hbm.at[idx0_v], r0_v, sem0)
        cp1 = pltpu.async_copy(ys_hbm.at[idx1_v], r1_v, sem1)
        cp0.wait()
        cp1.wait()

        def row(t, _):
            g0 = w0_v[t, :]
            g1 = w1_v[t, :]
            for v in range(DIM_ // 16):
                sl = pl.ds(v * 16, 16)
                r0_v[t, sl] = g0 * r0_v[t, sl] + g1 * r1_v[t, sl]
            return 0

        lax.fori_loop(0, sub, row, 0)
        pltpu.sync_copy(r0_v, y_hbm.at[pl.ds(tb, sub)])


def _combine(pos, w, ys):
    mesh = plsc.VectorSubcoreMesh(core_axis_name="c", subcore_axis_name="s",
                                  num_cores=NC_, num_subcores=NS_)
    return pl.kernel(
        _combine_body,
        out_type=jax.ShapeDtypeStruct((N_, DIM_), jnp.float32),
        mesh=mesh,
        scratch_types=[
            pltpu.VMEM((32,), jnp.int32),
            pltpu.VMEM((32,), jnp.int32),
            pltpu.VMEM((32, 16), jnp.float32),
            pltpu.VMEM((32, 16), jnp.float32),
            pltpu.VMEM((32, DIM_), jnp.float32),
            pltpu.VMEM((32, DIM_), jnp.float32),
            pltpu.SemaphoreType.DMA,
            pltpu.SemaphoreType.DMA,
        ],
    )(pos, w, ys)


def kernel(x, Wr, br, W1, b1, A1, B1, W2, b2, A2, B2):
    pos2, te, w, x16 = _route(x, Wr, br)
    pos = pos2.reshape(A_)
    x32 = lax.bitcast_convert_type(x16.reshape(N_, DIM_ // 2, 2), jnp.int32)
    xs32 = _dispatch(pos, x32)
    xs = lax.bitcast_convert_type(xs32, jnp.bfloat16).reshape(SPAD_, DIM_)
    ys = _ffn(te, xs, W1, b1, A1, B1, W2, b2, A2, B2)
    return _combine(pos, w, ys)


# speedup vs baseline: 2.1545x; 2.1545x over previous
import functools

import jax
import jax.numpy as jnp
from jax import lax
from jax.experimental import pallas as pl
from jax.experimental.pallas import tpu as pltpu
from jax.experimental.pallas import tpu_sc as plsc

DIM_ = 1024
HID_ = 2048
E_ = 8
R_ = 8
N_ = 2048
A_ = 2 * N_
TM_ = 512
SPAD_ = A_ + E_ * TM_
NT_ = SPAD_ // TM_
NC_ = 2
NS_ = 16
NW_ = NC_ * NS_

_CHUNK = 512


def _route_body(x_ref, wr_ref, br_ref, pos_ref, te_ref, w_ref):
    x = x_ref[...]
    wr = wr_ref[...]
    logits = lax.dot_general(wr, x, (((1,), (1,)), ((), ())),
                             preferred_element_type=jnp.float32)
    logits = logits + br_ref[...].reshape(E_, 1)
    iota_e = lax.broadcasted_iota(jnp.int32, (E_, N_), 0)
    neg_inf = jnp.float32(-jnp.inf)
    m1 = jnp.max(logits, axis=0, keepdims=True)
    i1 = jnp.min(jnp.where(logits == m1, iota_e, E_), axis=0, keepdims=True)
    masked = jnp.where(iota_e == i1, neg_inf, logits)
    m2 = jnp.max(masked, axis=0, keepdims=True)
    i2 = jnp.min(jnp.where(masked == m2, iota_e, E_), axis=0, keepdims=True)
    t = jnp.exp(m2 - m1)
    w1 = 1.0 / (1.0 + t)
    w2 = 1.0 - w1
    wcat = jnp.concatenate([w1, w2], axis=1)
    w_ref[...] = lax.dot_general(wcat, jnp.ones((1, 16), jnp.float32),
                                 (((0,), (0,)), ((), ())),
                                 preferred_element_type=jnp.float32)

    ef = jnp.concatenate([i1, i2], axis=1)
    hot = (lax.broadcasted_iota(jnp.int32, (E_, A_), 0) == ef
           ).astype(jnp.float32)
    tri = (lax.broadcasted_iota(jnp.int32, (_CHUNK, _CHUNK), 0)
           <= lax.broadcasted_iota(jnp.int32, (_CHUNK, _CHUNK), 1)
           ).astype(jnp.float32)
    carry = jnp.zeros((E_, 1), jnp.float32)
    chunks = []
    for c in range(A_ // _CHUNK):
        hc = lax.slice(hot, (0, c * _CHUNK), (E_, (c + 1) * _CHUNK))
        cc = lax.dot_general(hc, tri, (((1,), (0,)), ((), ())),
                             preferred_element_type=jnp.float32) + carry
        carry = lax.slice(cc, (0, _CHUNK - 1), (E_, _CHUNK))
        chunks.append(cc)
    csum = jnp.concatenate(chunks, axis=1)
    counts = carry
    tilecnt = jnp.floor((counts + (TM_ - 1)) * (1.0 / TM_))
    mstrict = (lax.broadcasted_iota(jnp.int32, (E_, E_), 1)
               < lax.broadcasted_iota(jnp.int32, (E_, E_), 0)
               ).astype(jnp.float32)
    tstart = lax.dot_general(mstrict, tilecnt, (((1,), (0,)), ((), ())),
                             preferred_element_type=jnp.float32)
    rank = jnp.sum(hot * csum, axis=0, keepdims=True) - 1.0
    base = jnp.sum(hot * (tstart * TM_), axis=0, keepdims=True)
    pos_ref[...] = (base + rank).astype(jnp.int32)

    tend = tstart + tilecnt
    iota_m = lax.broadcasted_iota(jnp.int32, (E_, 128), 1).astype(jnp.float32)
    te = jnp.sum((iota_m >= tend).astype(jnp.float32), axis=0,
                 keepdims=True)
    n_used = jnp.sum(tilecnt)
    iota_1m = lax.broadcasted_iota(jnp.int32, (1, 128), 1).astype(jnp.float32)
    fi = jnp.minimum(iota_1m, n_used - 1.0)
    valid = (iota_1m < n_used).astype(jnp.float32)
    te_ref[...] = jnp.concatenate(
        [jnp.minimum(te, E_ - 1), fi, valid], axis=0).astype(jnp.int32)


def _route(x, Wr, br):
    return pl.pallas_call(
        _route_body,
        out_shape=(
            jax.ShapeDtypeStruct((1, A_), jnp.int32),
            jax.ShapeDtypeStruct((3, 128), jnp.int32),
            jax.ShapeDtypeStruct((A_, 16), jnp.float32),
        ),
    )(x, Wr, br.reshape(1, E_))


_DSUB = 32
_DCH = (A_ // NW_) // _DSUB


def _dispatch_body(pos_hbm, x_hbm, xs_hbm, idx_v, rows0_v, rows1_v,
                   seml0, seml1, sems0, sems1):
    c = lax.axis_index("c")
    s = lax.axis_index("s")
    wid = s * NC_ + c
    per_w = A_ // NW_
    rows = [rows0_v, rows1_v]
    seml = [seml0, seml1]
    sems = [sems0, sems1]
    pltpu.sync_copy(pos_hbm.at[wid], idx_v)

    def tok(k):
        base = wid * per_w + k * _DSUB
        return jnp.where(base >= N_, base - N_, base)

    ld, sc = {}, {}
    ld[0] = pltpu.async_copy(x_hbm.at[pl.ds(tok(0), _DSUB)], rows[0], seml[0])
    for k in range(_DCH):
        b = k & 1
        nb = 1 - b
        if k + 1 < _DCH:
            if k >= 1:
                sc[k - 1].wait()
            ld[k + 1] = pltpu.async_copy(
                x_hbm.at[pl.ds(tok(k + 1), _DSUB)], rows[nb], seml[nb])
        ld[k].wait()
        sc[k] = pltpu.async_copy(rows[b], xs_hbm.at[idx_v.at[k]], sems[b])
    sc[_DCH - 2].wait()
    sc[_DCH - 1].wait()


def _dispatch(pos, x):
    mesh = plsc.VectorSubcoreMesh(core_axis_name="c", subcore_axis_name="s",
                                  num_cores=NC_, num_subcores=NS_)
    return pl.kernel(
        _dispatch_body,
        out_type=jax.ShapeDtypeStruct((SPAD_, DIM_), jnp.float32),
        mesh=mesh,
        scratch_types=[
            pltpu.VMEM((_DCH, _DSUB), jnp.int32),
            pltpu.VMEM((_DSUB, DIM_), jnp.float32),
            pltpu.VMEM((_DSUB, DIM_), jnp.float32),
            pltpu.SemaphoreType.DMA,
            pltpu.SemaphoreType.DMA,
            pltpu.SemaphoreType.DMA,
            pltpu.SemaphoreType.DMA,
        ],
    )(pos.reshape(NW_, _DCH, _DSUB), x)


def _ffn_body(te_ref, xs_ref, w1_ref, b1_ref, a1_ref, bb1_ref,
              w2_ref, b2_ref, a2_ref, bb2_ref, ys_ref,
              w1b_ref, w2b_ref, prev_ref):
    i = pl.program_id(0)
    e = te_ref[0, i]
    valid = te_ref[2, i] == 1

    nn = (((1,), (0,)), ((), ()))
    nt = (((1,), (1,)), ((), ()))

    @pl.when(valid & ((i == 0) | (e != prev_ref[0])))
    def _fold_weights():
        w1b_ref[...] = (w1_ref[0] + lax.dot_general(
            bb1_ref[0], a1_ref[0], nn, preferred_element_type=jnp.float32)
        ).astype(jnp.bfloat16)
        w2b_ref[...] = (w2_ref[0] + lax.dot_general(
            bb2_ref[0], a2_ref[0], nn, preferred_element_type=jnp.float32)
        ).astype(jnp.bfloat16)
        prev_ref[0] = e

    @pl.when(valid)
    def _compute():
        xb = xs_ref[...]
        u = lax.dot_general(xb.astype(jnp.bfloat16), w1b_ref[...],
                            nt, preferred_element_type=jnp.float32)
        u = u + b1_ref[0]
        h = u * (0.5 + 0.5 * lax.erf(u * jnp.float32(0.7071067811865476)))
        o = lax.dot_general(h.astype(jnp.bfloat16), w2b_ref[...],
                            nt, preferred_element_type=jnp.float32)
        ys_ref[...] = o + b2_ref[0]


def _ffn(te, xs, W1, b1, A1, B1, W2, b2, A2, B2):
    def em(i, te_ref):
        return (te_ref[0, i], 0, 0)

    grid_spec = pltpu.PrefetchScalarGridSpec(
        num_scalar_prefetch=1,
        grid=(NT_,),
        in_specs=[
            pl.BlockSpec((TM_, DIM_), lambda i, te_ref: (te_ref[1, i], 0)),
            pl.BlockSpec((1, HID_, DIM_), em),
            pl.BlockSpec((1, 1, HID_), em),
            pl.BlockSpec((1, R_, DIM_), em),
            pl.BlockSpec((1, HID_, R_), em),
            pl.BlockSpec((1, DIM_, HID_), em),
            pl.BlockSpec((1, 1, DIM_), em),
            pl.BlockSpec((1, R_, HID_), em),
            pl.BlockSpec((1, DIM_, R_), em),
        ],
        out_specs=pl.BlockSpec((TM_, DIM_), lambda i, te_ref: (i, 0)),
        scratch_shapes=[
            pltpu.VMEM((HID_, DIM_), jnp.bfloat16),
            pltpu.VMEM((DIM_, HID_), jnp.bfloat16),
            pltpu.SMEM((1,), jnp.int32),
        ],
    )
    return pl.pallas_call(
        _ffn_body,
        grid_spec=grid_spec,
        out_shape=jax.ShapeDtypeStruct((SPAD_, DIM_), jnp.float32),
    )(te, xs, W1, b1.reshape(E_, 1, HID_), A1, B1, W2,
      b2.reshape(E_, 1, DIM_), A2, B2)


_CSUB = 16
_CCH = (N_ // NW_) // _CSUB


def _combine_body(pos_hbm, w_hbm, ys_hbm, y_hbm,
                  posv0_v, posv1_v, w0s_v, w1s_v,
                  r00_v, r01_v, r10_v, r11_v,
                  sg00, sg01, sg10, sg11, st0, st1):
    c = lax.axis_index("c")
    s = lax.axis_index("s")
    wid = s * NC_ + c
    per_w = N_ // NW_
    r0 = [r00_v, r01_v]
    r1 = [r10_v, r11_v]
    sg0 = [sg00, sg01]
    sg1 = [sg10, sg11]
    sst = [st0, st1]
    pltpu.sync_copy(pos_hbm.at[0, wid], posv0_v)
    pltpu.sync_copy(pos_hbm.at[1, wid], posv1_v)
    pltpu.sync_copy(w_hbm.at[0, wid], w0s_v)
    pltpu.sync_copy(w_hbm.at[1, wid], w1s_v)

    g0, g1, st = {}, {}, {}
    g0[0] = pltpu.async_copy(ys_hbm.at[posv0_v.at[0]], r0[0], sg0[0])
    g1[0] = pltpu.async_copy(ys_hbm.at[posv1_v.at[0]], r1[0], sg1[0])
    for k in range(_CCH):
        b = k & 1
        nb = 1 - b
        if k + 1 < _CCH:
            if k >= 1:
                st[k - 1].wait()
            g0[k + 1] = pltpu.async_copy(
                ys_hbm.at[posv0_v.at[k + 1]], r0[nb], sg0[nb])
            g1[k + 1] = pltpu.async_copy(
                ys_hbm.at[posv1_v.at[k + 1]], r1[nb], sg1[nb])
        g0[k].wait()
        g1[k].wait()

        def row(t, _):
            gv0 = w0s_v[k, t, :]
            gv1 = w1s_v[k, t, :]
            for v in range(DIM_ // 16):
                sl = pl.ds(v * 16, 16)
                r0[b][t, sl] = gv0 * r0[b][t, sl] + gv1 * r1[b][t, sl]
            return 0

        lax.fori_loop(0, _CSUB, row, 0)
        st[k] = pltpu.async_copy(
            r0[b], y_hbm.at[pl.ds(wid * per_w + k * _CSUB, _CSUB)], sst[b])
    st[_CCH - 2].wait()
    st[_CCH - 1].wait()


def _combine(pos, w, ys):
    mesh = plsc.VectorSubcoreMesh(core_axis_name="c", subcore_axis_name="s",
                                  num_cores=NC_, num_subcores=NS_)
    return pl.kernel(
        _combine_body,
        out_type=jax.ShapeDtypeStruct((N_, DIM_), jnp.float32),
        mesh=mesh,
        scratch_types=[
            pltpu.VMEM((_CCH, _CSUB), jnp.int32),
            pltpu.VMEM((_CCH, _CSUB), jnp.int32),
            pltpu.VMEM((_CCH, _CSUB, 16), jnp.float32),
            pltpu.VMEM((_CCH, _CSUB, 16), jnp.float32),
            pltpu.VMEM((_CSUB, DIM_), jnp.float32),
            pltpu.VMEM((_CSUB, DIM_), jnp.float32),
            pltpu.VMEM((_CSUB, DIM_), jnp.float32),
            pltpu.VMEM((_CSUB, DIM_), jnp.float32),
            pltpu.SemaphoreType.DMA,
            pltpu.SemaphoreType.DMA,
            pltpu.SemaphoreType.DMA,
            pltpu.SemaphoreType.DMA,
            pltpu.SemaphoreType.DMA,
            pltpu.SemaphoreType.DMA,
        ],
    )(pos.reshape(2, NW_, _CCH, _CSUB), w.reshape(2, NW_, _CCH, _CSUB, 16),
      ys)


def kernel(x, Wr, br, W1, b1, A1, B1, W2, b2, A2, B2):
    pos2, te, w = _route(x, Wr, br)
    pos = pos2.reshape(A_)
    xs = _dispatch(pos, x)
    ys = _ffn(te, xs, W1, b1, A1, B1, W2, b2, A2, B2)
    return _combine(pos, w, ys)
